# baseline (speedup 1.0000x reference)
import math
from functools import partial

import jax
import jax.numpy as jnp
from jax.experimental import pallas as pl
from jax.experimental.pallas import tpu as pltpu

_INV_SQRT2 = 0.7071067811865476


def _ln_f32(x, g, b, eps=1e-5):
    x = x.astype(jnp.float32)
    mu = jnp.mean(x, axis=-1, keepdims=True)
    ex2 = jnp.mean(x * x, axis=-1, keepdims=True)
    var = ex2 - mu * mu
    return (x - mu) * jax.lax.rsqrt(var + eps) * g + b


def _encoder_kernel(x_ref, g1_ref, b1_ref, wqkv_ref, bqkv_ref,
                    wo_ref, bo_ref, g2_ref, b2_ref,
                    w1_ref, fb1_ref, w2_ref, fb2_ref, o_ref,
                    *, num_heads):
    xb = x_ref[0]
    S, E = xb.shape
    dh = E // num_heads

    x2 = _ln_f32(xb, g1_ref[...], b1_ref[...])
    x2b = x2.astype(jnp.bfloat16)
    qkv_parts = []
    for c in range(3):
        pc = jnp.dot(x2b, wqkv_ref[:, c * E:(c + 1) * E],
                     preferred_element_type=jnp.float32) + bqkv_ref[:, c * E:(c + 1) * E]
        qkv_parts.append(pc.astype(jnp.bfloat16))
    qkvb = jnp.concatenate(qkv_parts, axis=1)

    heads = []
    for h in range(num_heads):
        qh = qkvb[:, h * dh:(h + 1) * dh]
        kh = qkvb[:, E + h * dh:E + (h + 1) * dh]
        vh = qkvb[:, 2 * E + h * dh:2 * E + (h + 1) * dh]
        s = jax.lax.dot_general(qh, kh, (((1,), (1,)), ((), ())),
                                preferred_element_type=jnp.float32)
        m = jnp.max(s, axis=-1, keepdims=True)
        p = jnp.exp2(s - m)
        l = jnp.sum(p, axis=-1, keepdims=True)
        pv = jnp.dot(p.astype(jnp.bfloat16), vh,
                     preferred_element_type=jnp.float32)
        heads.append((pv * (1.0 / l)).astype(jnp.bfloat16))
    attn = jnp.concatenate(heads, axis=1)
    proj = jnp.dot(attn, wo_ref[...], preferred_element_type=jnp.float32)

    y = xb + proj + bo_ref[...]

    x2 = _ln_f32(y, g2_ref[...], b2_ref[...])
    x2b = x2.astype(jnp.bfloat16)
    F = w1_ref.shape[1]
    n_chunks = 8
    fc = F // n_chunks
    h1_parts = []
    for c in range(n_chunks):
        h1c = jnp.dot(x2b, w1_ref[:, c * fc:(c + 1) * fc],
                      preferred_element_type=jnp.float32) + fb1_ref[:, c * fc:(c + 1) * fc]
        h1c = 0.5 * h1c * (1.0 + jax.lax.erf(h1c * _INV_SQRT2))
        h1_parts.append(h1c.astype(jnp.bfloat16))
    h1b = jnp.concatenate(h1_parts, axis=1)
    ff = jnp.dot(h1b, w2_ref[...],
                 preferred_element_type=jnp.float32) + fb2_ref[...]
    o_ref[0] = (y + ff).astype(o_ref.dtype)


def _encoder_shard(x, ln1_g, ln1_b, wq, bq, wk, bk, wv, bv, wo, bo,
                   ln2_g, ln2_b, w1, b1, w2, b2, *, num_heads, qscale):
    B, S, E = x.shape
    F = w1.shape[1]
    wqkv = jnp.concatenate([wq * qscale, wk, wv], axis=1).astype(jnp.bfloat16)
    bqkv = jnp.concatenate([bq * qscale, bk, bv], axis=1)
    wo_b = wo.astype(jnp.bfloat16)
    w1_b = w1.astype(jnp.bfloat16)
    w2_b = w2.astype(jnp.bfloat16)

    const = lambda b: (0, 0)
    return pl.pallas_call(
        partial(_encoder_kernel, num_heads=num_heads),
        out_shape=jax.ShapeDtypeStruct((B, S, E), x.dtype),
        grid=(B,),
        in_specs=[
            pl.BlockSpec((1, S, E), lambda b: (b, 0, 0)),
            pl.BlockSpec((1, E), const),
            pl.BlockSpec((1, E), const),
            pl.BlockSpec((E, 3 * E), const),
            pl.BlockSpec((1, 3 * E), const),
            pl.BlockSpec((E, E), const),
            pl.BlockSpec((1, E), const),
            pl.BlockSpec((1, E), const),
            pl.BlockSpec((1, E), const),
            pl.BlockSpec((E, F), const),
            pl.BlockSpec((1, F), const),
            pl.BlockSpec((F, E), const),
            pl.BlockSpec((1, E), const),
        ],
        out_specs=pl.BlockSpec((1, S, E), lambda b: (b, 0, 0)),
        compiler_params=pltpu.CompilerParams(
            dimension_semantics=("parallel",),
            vmem_limit_bytes=64 << 20),
    )(x, ln1_g, ln1_b, wqkv, bqkv, wo_b, bo, ln2_g, ln2_b, w1_b, b1, w2_b, b2)


def kernel(x, ln1_g, ln1_b, wq, bq, wk, bk, wv, bv, wo, bo, ln2_g, ln2_b,
           w1, b1, w2, b2):
    B, S, E = x.shape
    H = 8
    dh = E // H
    scale = 1.0 / math.sqrt(dh)
    qscale = scale * math.log2(math.e)

    devs = jax.devices()
    ndev = 1
    for cand in (8, 4, 2):
        if len(devs) >= cand:
            ndev = cand
            break
    mesh = jax.sharding.Mesh(devs[:ndev], ("d",))
    P = jax.sharding.PartitionSpec
    rep2 = P(None, None)
    fn = jax.shard_map(
        partial(_encoder_shard, num_heads=H, qscale=qscale),
        mesh=mesh,
        in_specs=(P("d", None, None),) + (rep2,) * 16,
        out_specs=P("d", None, None),
        check_vma=False,
    )
    return fn(x, ln1_g, ln1_b, wq, bq, wk, bk, wv, bv, wo, bo, ln2_g, ln2_b,
              w1, b1, w2, b2)


# speedup vs baseline: 4.8440x; 4.8440x over previous
import math
from functools import partial

import jax
import jax.numpy as jnp
from jax.experimental import pallas as pl
from jax.experimental.pallas import tpu as pltpu

_INV_SQRT2 = 0.7071067811865476


def _ln_f32(x, g, b, eps=1e-5):
    x = x.astype(jnp.float32)
    mu = jnp.mean(x, axis=-1, keepdims=True)
    ex2 = jnp.mean(x * x, axis=-1, keepdims=True)
    var = ex2 - mu * mu
    return (x - mu) * jax.lax.rsqrt(var + eps) * g + b


def _encoder_kernel(x_ref, g1_ref, b1_ref, wqkv_ref, bqkv_ref,
                    wo_ref, bo_ref, g2_ref, b2_ref,
                    w1_ref, fb1_ref, w2_ref, fb2_ref, o_ref,
                    *, num_heads):
    xb = x_ref[0]
    S, E = xb.shape
    dh = E // num_heads

    x2 = _ln_f32(xb, g1_ref[...], b1_ref[...])
    x2b = x2.astype(jnp.bfloat16)
    qkv_parts = []
    for c in range(3):
        pc = jnp.dot(x2b, wqkv_ref[:, c * E:(c + 1) * E],
                     preferred_element_type=jnp.float32) + bqkv_ref[:, c * E:(c + 1) * E]
        qkv_parts.append(pc.astype(jnp.bfloat16))
    qkvb = jnp.concatenate(qkv_parts, axis=1)

    heads = []
    for h in range(num_heads):
        qh = qkvb[:, h * dh:(h + 1) * dh]
        kh = qkvb[:, E + h * dh:E + (h + 1) * dh]
        vh = qkvb[:, 2 * E + h * dh:2 * E + (h + 1) * dh]
        s = jax.lax.dot_general(qh, kh, (((1,), (1,)), ((), ())),
                                preferred_element_type=jnp.float32)
        m = jnp.max(s, axis=-1, keepdims=True)
        p = jnp.exp2(s - m)
        l = jnp.sum(p, axis=-1, keepdims=True)
        pv = jnp.dot(p.astype(jnp.bfloat16), vh,
                     preferred_element_type=jnp.float32)
        heads.append((pv * (1.0 / l)).astype(jnp.bfloat16))
    attn = jnp.concatenate(heads, axis=1)
    proj = jnp.dot(attn, wo_ref[...], preferred_element_type=jnp.float32)

    y = xb + proj + bo_ref[...]

    x2 = _ln_f32(y, g2_ref[...], b2_ref[...])
    x2b = x2.astype(jnp.bfloat16)
    F = w1_ref.shape[1]
    n_chunks = 8
    fc = F // n_chunks
    h1_parts = []
    for c in range(n_chunks):
        h1c = jnp.dot(x2b, w1_ref[:, c * fc:(c + 1) * fc],
                      preferred_element_type=jnp.float32) + fb1_ref[:, c * fc:(c + 1) * fc]
        h1c = 0.5 * h1c * (1.0 + jax.lax.erf(h1c * _INV_SQRT2))
        h1_parts.append(h1c.astype(jnp.bfloat16))
    h1b = jnp.concatenate(h1_parts, axis=1)
    ff = jnp.dot(h1b, w2_ref[...],
                 preferred_element_type=jnp.float32) + fb2_ref[...]
    o_ref[0] = (y + ff).astype(o_ref.dtype)


def kernel(x, ln1_g, ln1_b, wq, bq, wk, bk, wv, bv, wo, bo, ln2_g, ln2_b,
           w1, b1, w2, b2):
    B, S, E = x.shape
    H = 8
    dh = E // H
    F = w1.shape[1]
    scale = 1.0 / math.sqrt(dh)

    qscale = scale * math.log2(math.e)
    wqkv = jnp.concatenate([wq * qscale, wk, wv], axis=1).astype(jnp.bfloat16)
    bqkv = jnp.concatenate([bq * qscale, bk, bv], axis=1)
    wo_b = wo.astype(jnp.bfloat16)
    w1_b = w1.astype(jnp.bfloat16)
    w2_b = w2.astype(jnp.bfloat16)

    const = lambda b: (0, 0)
    return pl.pallas_call(
        partial(_encoder_kernel, num_heads=H),
        out_shape=jax.ShapeDtypeStruct((B, S, E), x.dtype),
        grid=(B,),
        in_specs=[
            pl.BlockSpec((1, S, E), lambda b: (b, 0, 0)),
            pl.BlockSpec((1, E), const),
            pl.BlockSpec((1, E), const),
            pl.BlockSpec((E, 3 * E), const),
            pl.BlockSpec((1, 3 * E), const),
            pl.BlockSpec((E, E), const),
            pl.BlockSpec((1, E), const),
            pl.BlockSpec((1, E), const),
            pl.BlockSpec((1, E), const),
            pl.BlockSpec((E, F), const),
            pl.BlockSpec((1, F), const),
            pl.BlockSpec((F, E), const),
            pl.BlockSpec((1, E), const),
        ],
        out_specs=pl.BlockSpec((1, S, E), lambda b: (b, 0, 0)),
        compiler_params=pltpu.CompilerParams(
            dimension_semantics=("parallel",),
            vmem_limit_bytes=64 << 20),
    )(x, ln1_g, ln1_b, wqkv, bqkv, wo_b, bo, ln2_g, ln2_b, w1_b, b1, w2_b, b2)
